# baseline (speedup 1.0000x reference)
import jax
import jax.numpy as jnp
from jax import lax
from jax.experimental import pallas as pl
from jax.experimental.pallas import tpu as pltpu
from jax.experimental.pallas import tpu_sc as plsc

MS = 14
SP = MS * MS


KSLAB = 14
NSTEP = SP // KSLAB


def _tree_sum(terms):
    while len(terms) > 1:
        nxt = [a + b for a, b in zip(terms[::2], terms[1::2])]
        if len(terms) % 2:
            nxt.append(terms[-1])
        terms = nxt
    return terms[0]


def _masked_mean_kernel(coords_ref, u_ref, s_ref, o_ref, b_ref,
                        acc_s, acc_o, acc_b):
    i = pl.program_id(0)
    c = coords_ref[...]
    x0, x1 = c[:, 0:1], c[:, 1:2]
    ox0, ox1 = c[:, 2:3], c[:, 3:4]
    y0, y1 = c[:, 4:5], c[:, 5:6]
    oy0, oy1 = c[:, 6:7], c[:, 7:8]

    ts = acc_s[...]
    to = acc_o[...]
    tb = acc_b[...]
    zero = jnp.zeros_like(ts)
    if_first = i == 0
    ts = jnp.where(if_first, 0.0, ts)
    to = jnp.where(if_first, 0.0, to)
    tb = jnp.where(if_first, 0.0, tb)

    rf = (i // (MS // KSLAB)).astype(jnp.float32)
    cbase = (i % (MS // KSLAB)) * KSLAB
    rin_s = (rf >= x0) & (rf < x1)
    rin_o = (rf >= ox0) & (rf < ox1)
    cdim = ts.shape[1]
    sel = (lax.broadcasted_iota(jnp.int32, (3, 3 * cdim), 0)
           == lax.broadcasted_iota(jnp.int32, (3, 3 * cdim), 1) // cdim
           ).astype(jnp.float32)
    terms_s, terms_o, terms_b = [], [], []
    for k in range(KSLAB):
        cf = (cbase + k).astype(jnp.float32)
        msk = (rin_s & (cf >= y0) & (cf < y1)).astype(jnp.float32)
        mok = (rin_o & (cf >= oy0) & (cf < oy1)).astype(jnp.float32)
        mbk = jnp.maximum(1.0 - msk - mok, 0.0)
        mvec = jnp.concatenate([msk, mok, mbk], axis=1)
        bc = lax.dot_general(mvec, sel, (((1,), (0,)), ((), ())),
                             preferred_element_type=jnp.float32)
        u = u_ref[0, k]
        terms_s.append(u * bc[:, 0:cdim])
        terms_o.append(u * bc[:, cdim:2 * cdim])
        terms_b.append(u * bc[:, 2 * cdim:3 * cdim])
    ts = ts + _tree_sum(terms_s)
    to = to + _tree_sum(terms_o)
    tb = tb + _tree_sum(terms_b)

    acc_s[...] = ts
    acc_o[...] = to
    acc_b[...] = tb

    @pl.when(i == NSTEP - 1)
    def _():
        inv = 1.0 / SP
        s_ref[...] = ts * inv
        o_ref[...] = to * inv
        b_ref[...] = tb * inv


def _mm(a, b):
    return lax.dot_general(a, b, (((1,), (0,)), ((), ())),
                           preferred_element_type=jnp.float32)


def _mm_t(a, b):
    return lax.dot_general(a, b, (((0,), (0,)), ((), ())),
                           preferred_element_type=jnp.float32)


def _obj_embed_kernel(roi_ref, logits_ref, bboxes_ref,
                      w1a_ref, w1b_ref, w1c_ref, be1_ref, we2_ref, be2_ref,
                      wg_ref, objf_ref, gobj_ref):
    h1 = _mm(roi_ref[...], w1a_ref[...]) + _mm(logits_ref[...], w1b_ref[...]) \
        + _mm(bboxes_ref[...], w1c_ref[...]) + be1_ref[...]
    objf = _mm(jnp.maximum(h1, 0.0), we2_ref[...]) + be2_ref[...]
    objf_ref[...] = objf
    gobj_ref[...] = _mm(objf, wg_ref[...])


SC_W = 128
SC_LANES = 16


def _pair_gather_sc(g_obj, sub_idx, obj_idx):
    mp = sub_idx.shape[1]
    c = g_obj.shape[1]
    mesh = plsc.VectorSubcoreMesh(core_axis_name="c", subcore_axis_name="s",
                                  num_cores=2, num_subcores=16)

    @pl.kernel(out_type=jax.ShapeDtypeStruct((mp, c), jnp.float32),
               mesh=mesh,
               scratch_types=[pltpu.VMEM((SC_W, c), jnp.float32)])
    def kern(g_hbm, isub_hbm, iobj_hbm, o_hbm, scr_b):
        def body(isub_vmem, iobj_vmem, o_vmem):
            pltpu.sync_copy(g_hbm.at[isub_vmem.at[0]], o_vmem)
            pltpu.sync_copy(g_hbm.at[iobj_vmem.at[0]], scr_b)

            @pl.loop(0, SC_W)
            def _(rr):
                @pl.loop(0, c, step=SC_LANES)
                def _(cc):
                    slc = (pl.ds(rr, 1), pl.ds(cc, SC_LANES))
                    o_vmem.at[*slc][...] = o_vmem.at[*slc][...] + scr_b.at[*slc][...]

        pltpu.emit_pipeline(
            body,
            grid=(mp // SC_W,),
            in_specs=[pl.BlockSpec((1, SC_W), lambda i: (0, i)),
                      pl.BlockSpec((1, SC_W), lambda i: (0, i))],
            out_specs=[pl.BlockSpec((SC_W, c), lambda i: (i, 0))],
            core_axis_name=("c", "s"),
            dimension_semantics=(pltpu.PARALLEL,),
        )(isub_hbm, iobj_hbm, o_hbm)

    return kern(g_obj, sub_idx, obj_idx)


def _fuse_kernel(pairs_ref,
                 ss_ref, so_ref, sb_ref,
                 objf_ref, gobj_ref, msg_ref,
                 wsr_ref, bsr_ref, wsu_ref, bsu_ref,
                 wor_ref, bor_ref, wou_ref, bou_ref,
                 wbr_ref, bbr_ref, wbu_ref, bbu_ref,
                 wc1a_ref, wc1b_ref, wc1c_ref, bc1_ref, wc2_ref, bc2_ref,
                 wg_ref, bgc_ref, wobj_ref, bobj_ref, wrel_ref, brel_ref,
                 objd_ref, reld_ref):
    f32 = jnp.float32
    mm = _mm
    mm_t = _mm_t
    obj_feats = objf_ref[...]

    def gate(s, wr, br, wu, bu):
        a = jax.nn.sigmoid(mm(jnp.maximum(mm(s, wr) + br, 0.0), wu) + bu)
        return s * a

    vs = gate(ss_ref[...], wsr_ref[...], bsr_ref[...], wsu_ref[...], bsu_ref[...])
    vo = gate(so_ref[...], wor_ref[...], bor_ref[...], wou_ref[...], bou_ref[...])
    vb = gate(sb_ref[...], wbr_ref[...], bbr_ref[...], wbu_ref[...], bbu_ref[...])

    rh = jnp.maximum(mm(vs, wc1a_ref[...]) + mm(vo, wc1b_ref[...])
                     + mm(vb, wc1c_ref[...]) + bc1_ref[...], 0.0)
    rel_feats = mm(rh, wc2_ref[...]) + bc2_ref[...]

    n = objf_ref.shape[0]
    m = rel_feats.shape[0]
    pairs = pairs_ref[...]
    obj_ids = lax.broadcasted_iota(jnp.int32, (m, n), 1)
    s_hot = (pairs[:, 0:1] == obj_ids).astype(f32)
    o_hot = (pairs[:, 1:2] == obj_ids).astype(f32)
    so = s_hot + o_hot

    g_obj = gobj_ref[...]
    g_rel = mm(rel_feats, wg_ref[...])

    a_oo = mm_t(s_hot, o_hot)
    agg_obj = mm(a_oo, g_obj) + mm_t(so, g_rel) + g_obj
    deg_obj = 1.0 + jnp.sum(a_oo, axis=1, keepdims=True) \
        + jnp.sum(so, axis=0)[:, None]
    h_obj = jnp.maximum(agg_obj / deg_obj + bgc_ref[...], 0.0)

    agg_rel = msg_ref[...] + g_rel
    h_rel = jnp.maximum(agg_rel * (1.0 / 3.0) + bgc_ref[...], 0.0)

    out_obj = h_obj + obj_feats
    out_rel = h_rel + rel_feats
    objd_ref[...] = mm(out_obj, wobj_ref[...]) + bobj_ref[...]
    reld_ref[...] = mm(out_rel, wrel_ref[...]) + brel_ref[...]


def kernel(roi_features, obj_logits, bboxes, union_features, rel_pair_idxs,
           We1, be1, We2, be2,
           Wsr, bsr, Wsu, bsu, Wor, bor, Wou, bou, Wbr, bbr, Wbu, bbu,
           Wc1, bc1, Wc2, bc2, Wg, bgc, Wobj, bobj, Wrel, brel):
    f32 = jnp.float32
    n, roi = roi_features.shape
    m, c = union_features.shape[0], union_features.shape[1]
    objc = obj_logits.shape[1]
    relc = Wrel.shape[1]

    sb = bboxes[rel_pair_idxs[:, 0]]
    ob = bboxes[rel_pair_idxs[:, 1]]
    pair_boxes = jnp.concatenate([sb, ob], axis=1)
    union_boxes = jnp.concatenate(
        [jnp.minimum(sb[:, :2], ob[:, :2]), jnp.maximum(sb[:, 2:], ob[:, 2:])], axis=1)
    x = pair_boxes[:, jnp.array([0, 2, 4, 6])] - union_boxes[:, 0:1]
    y = pair_boxes[:, jnp.array([1, 3, 5, 7])] - union_boxes[:, 1:2]
    xr = MS / jnp.maximum(x[:, 1], x[:, 3])
    yr = MS / jnp.maximum(y[:, 1], y[:, 3])
    xp = jnp.clip(jnp.round(x * xr[:, None]), 0, MS)
    yp = jnp.clip(jnp.round(y * yr[:, None]), 0, MS)
    coords = jnp.concatenate([xp, yp], axis=1).astype(f32)

    row = lambda v: v.reshape(1, -1)
    w1a = We1[:roi]
    w1b = We1[roi:roi + objc]
    w1c = We1[roi + objc:]
    wc1a = Wc1[:c]
    wc1b = Wc1[c:2 * c]
    wc1c = Wc1[2 * c:]

    obj_feats, g_obj = pl.pallas_call(
        _obj_embed_kernel,
        out_shape=[jax.ShapeDtypeStruct((n, c), f32),
                   jax.ShapeDtypeStruct((n, c), f32)],
    )(roi_features, obj_logits, bboxes,
      w1a, w1b, w1c, row(be1), We2, row(be2), Wg)

    mp = ((m + SC_W - 1) // SC_W) * SC_W
    sub_p = jnp.zeros((1, mp), jnp.int32).at[0, :m].set(rel_pair_idxs[:, 0])
    obj_p = jnp.zeros((1, mp), jnp.int32).at[0, :m].set(rel_pair_idxs[:, 1])
    msg = _pair_gather_sc(g_obj, sub_p, obj_p)[:m]

    ut = jnp.transpose(union_features, (2, 3, 0, 1))
    ss, so, sbg = pl.pallas_call(
        _masked_mean_kernel,
        grid=(NSTEP,),
        in_specs=[
            pl.BlockSpec((m, 8), lambda i: (0, 0)),
            pl.BlockSpec((1, KSLAB, m, c),
                         lambda i: (i // (MS // KSLAB), i % (MS // KSLAB), 0, 0)),
        ],
        out_specs=[
            pl.BlockSpec((m, c), lambda i: (0, 0)),
            pl.BlockSpec((m, c), lambda i: (0, 0)),
            pl.BlockSpec((m, c), lambda i: (0, 0)),
        ],
        out_shape=[jax.ShapeDtypeStruct((m, c), f32)] * 3,
        scratch_shapes=[pltpu.VMEM((m, c), f32)] * 3,
    )(coords, ut)

    obj_dists, rel_dists = pl.pallas_call(
        _fuse_kernel,
        out_shape=[jax.ShapeDtypeStruct((n, objc), f32),
                   jax.ShapeDtypeStruct((m, relc), f32)],
    )(rel_pair_idxs,
      ss, so, sbg,
      obj_feats, g_obj, msg,
      Wsr, row(bsr), Wsu, row(bsu),
      Wor, row(bor), Wou, row(bou),
      Wbr, row(bbr), Wbu, row(bbu),
      wc1a, wc1b, wc1c, row(bc1), Wc2, row(bc2),
      Wg, row(bgc), Wobj, row(bobj), Wrel, row(brel))
    return (obj_dists, rel_dists)


# speedup vs baseline: 1.0147x; 1.0147x over previous
import jax
import jax.numpy as jnp
from jax import lax
from jax.experimental import pallas as pl
from jax.experimental.pallas import tpu as pltpu
from jax.experimental.pallas import tpu_sc as plsc

MS = 14
SP = MS * MS


KSLAB = 14
NSTEP = SP // KSLAB


def _tree_sum(terms):
    while len(terms) > 1:
        nxt = [a + b for a, b in zip(terms[::2], terms[1::2])]
        if len(terms) % 2:
            nxt.append(terms[-1])
        terms = nxt
    return terms[0]


def _masked_mean_kernel(coords_ref, u_ref, s_ref, o_ref, b_ref,
                        acc_s, acc_o, acc_b):
    i = pl.program_id(0)
    c = coords_ref[...]
    x0, x1 = c[:, 0:1], c[:, 1:2]
    ox0, ox1 = c[:, 2:3], c[:, 3:4]
    y0, y1 = c[:, 4:5], c[:, 5:6]
    oy0, oy1 = c[:, 6:7], c[:, 7:8]

    ts = acc_s[...]
    to = acc_o[...]
    tb = acc_b[...]
    zero = jnp.zeros_like(ts)
    if_first = i == 0
    ts = jnp.where(if_first, 0.0, ts)
    to = jnp.where(if_first, 0.0, to)
    tb = jnp.where(if_first, 0.0, tb)

    rf = (i // (MS // KSLAB)).astype(jnp.float32)
    cbase = (i % (MS // KSLAB)) * KSLAB
    rin_s = (rf >= x0) & (rf < x1)
    rin_o = (rf >= ox0) & (rf < ox1)
    cdim = ts.shape[1]
    sel = (lax.broadcasted_iota(jnp.int32, (3, 3 * cdim), 0)
           == lax.broadcasted_iota(jnp.int32, (3, 3 * cdim), 1) // cdim
           ).astype(jnp.float32)
    terms_s, terms_o, terms_b = [], [], []
    for k in range(KSLAB):
        cf = (cbase + k).astype(jnp.float32)
        msk = (rin_s & (cf >= y0) & (cf < y1)).astype(jnp.float32)
        mok = (rin_o & (cf >= oy0) & (cf < oy1)).astype(jnp.float32)
        mbk = jnp.maximum(1.0 - msk - mok, 0.0)
        mvec = jnp.concatenate([msk, mok, mbk], axis=1)
        bc = lax.dot_general(mvec, sel, (((1,), (0,)), ((), ())),
                             preferred_element_type=jnp.float32)
        u = u_ref[0, k]
        terms_s.append(u * bc[:, 0:cdim])
        terms_o.append(u * bc[:, cdim:2 * cdim])
        terms_b.append(u * bc[:, 2 * cdim:3 * cdim])
    ts = ts + _tree_sum(terms_s)
    to = to + _tree_sum(terms_o)
    tb = tb + _tree_sum(terms_b)

    acc_s[...] = ts
    acc_o[...] = to
    acc_b[...] = tb

    @pl.when(i == NSTEP - 1)
    def _():
        inv = 1.0 / SP
        s_ref[...] = ts * inv
        o_ref[...] = to * inv
        b_ref[...] = tb * inv


def _mm(a, b):
    return lax.dot_general(a, b, (((1,), (0,)), ((), ())),
                           preferred_element_type=jnp.float32)


def _mm_t(a, b):
    return lax.dot_general(a, b, (((0,), (0,)), ((), ())),
                           preferred_element_type=jnp.float32)


def _obj_embed_kernel(roi_ref, logits_ref, bboxes_ref,
                      w1a_ref, w1b_ref, w1c_ref, be1_ref, we2_ref, be2_ref,
                      wg_ref, objf_ref, gobj_ref):
    h1 = _mm(roi_ref[...], w1a_ref[...]) + _mm(logits_ref[...], w1b_ref[...]) \
        + _mm(bboxes_ref[...], w1c_ref[...]) + be1_ref[...]
    objf = _mm(jnp.maximum(h1, 0.0), we2_ref[...]) + be2_ref[...]
    objf_ref[...] = objf
    gobj_ref[...] = _mm(objf, wg_ref[...])


SC_W = 128
SC_LANES = 16


SC_ROW = 128


def _pair_gather_sc(g_rows, idx):
    k = idx.shape[1]
    mesh = plsc.VectorSubcoreMesh(core_axis_name="c", subcore_axis_name="s",
                                  num_cores=2, num_subcores=16)

    @pl.kernel(out_type=jax.ShapeDtypeStruct((k, SC_ROW), jnp.float32),
               mesh=mesh)
    def kern(g_hbm, idx_hbm, o_hbm):
        def body(idx_vmem, o_vmem):
            pltpu.sync_copy(g_hbm.at[idx_vmem.at[0]], o_vmem)

        pltpu.emit_pipeline(
            body,
            grid=(k // SC_W,),
            in_specs=[pl.BlockSpec((1, SC_W), lambda i: (0, i))],
            out_specs=[pl.BlockSpec((SC_W, SC_ROW), lambda i: (i, 0))],
            core_axis_name=("c", "s"),
            dimension_semantics=(pltpu.PARALLEL,),
        )(idx_hbm, o_hbm)

    return kern(g_rows, idx)


def _fuse_kernel(pairs_ref,
                 ss_ref, so_ref, sb_ref,
                 objf_ref, gobj_ref, msga_ref, msgb_ref,
                 wsr_ref, bsr_ref, wsu_ref, bsu_ref,
                 wor_ref, bor_ref, wou_ref, bou_ref,
                 wbr_ref, bbr_ref, wbu_ref, bbu_ref,
                 wc1a_ref, wc1b_ref, wc1c_ref, bc1_ref, wc2_ref, bc2_ref,
                 wg_ref, bgc_ref, wobj_ref, bobj_ref, wrel_ref, brel_ref,
                 objd_ref, reld_ref):
    f32 = jnp.float32
    mm = _mm
    mm_t = _mm_t
    obj_feats = objf_ref[...]

    def gate(s, wr, br, wu, bu):
        a = jax.nn.sigmoid(mm(jnp.maximum(mm(s, wr) + br, 0.0), wu) + bu)
        return s * a

    vs = gate(ss_ref[...], wsr_ref[...], bsr_ref[...], wsu_ref[...], bsu_ref[...])
    vo = gate(so_ref[...], wor_ref[...], bor_ref[...], wou_ref[...], bou_ref[...])
    vb = gate(sb_ref[...], wbr_ref[...], bbr_ref[...], wbu_ref[...], bbu_ref[...])

    rh = jnp.maximum(mm(vs, wc1a_ref[...]) + mm(vo, wc1b_ref[...])
                     + mm(vb, wc1c_ref[...]) + bc1_ref[...], 0.0)
    rel_feats = mm(rh, wc2_ref[...]) + bc2_ref[...]

    n = objf_ref.shape[0]
    m = rel_feats.shape[0]
    pairs = pairs_ref[...]
    obj_ids = lax.broadcasted_iota(jnp.int32, (m, n), 1)
    s_hot = (pairs[:, 0:1] == obj_ids).astype(f32)
    o_hot = (pairs[:, 1:2] == obj_ids).astype(f32)
    so = s_hot + o_hot

    g_obj = gobj_ref[...]
    g_rel = mm(rel_feats, wg_ref[...])

    a_oo = mm_t(s_hot, o_hot)
    agg_obj = mm(a_oo, g_obj) + mm_t(so, g_rel) + g_obj
    deg_obj = 1.0 + jnp.sum(a_oo, axis=1, keepdims=True) \
        + jnp.sum(so, axis=0)[:, None]
    h_obj = jnp.maximum(agg_obj / deg_obj + bgc_ref[...], 0.0)

    agg_rel = msga_ref[...] + msgb_ref[...] + g_rel
    h_rel = jnp.maximum(agg_rel * (1.0 / 3.0) + bgc_ref[...], 0.0)

    out_obj = h_obj + obj_feats
    out_rel = h_rel + rel_feats
    objd_ref[...] = mm(out_obj, wobj_ref[...]) + bobj_ref[...]
    reld_ref[...] = mm(out_rel, wrel_ref[...]) + brel_ref[...]


def kernel(roi_features, obj_logits, bboxes, union_features, rel_pair_idxs,
           We1, be1, We2, be2,
           Wsr, bsr, Wsu, bsu, Wor, bor, Wou, bou, Wbr, bbr, Wbu, bbu,
           Wc1, bc1, Wc2, bc2, Wg, bgc, Wobj, bobj, Wrel, brel):
    f32 = jnp.float32
    n, roi = roi_features.shape
    m, c = union_features.shape[0], union_features.shape[1]
    objc = obj_logits.shape[1]
    relc = Wrel.shape[1]

    sb = bboxes[rel_pair_idxs[:, 0]]
    ob = bboxes[rel_pair_idxs[:, 1]]
    pair_boxes = jnp.concatenate([sb, ob], axis=1)
    union_boxes = jnp.concatenate(
        [jnp.minimum(sb[:, :2], ob[:, :2]), jnp.maximum(sb[:, 2:], ob[:, 2:])], axis=1)
    x = pair_boxes[:, jnp.array([0, 2, 4, 6])] - union_boxes[:, 0:1]
    y = pair_boxes[:, jnp.array([1, 3, 5, 7])] - union_boxes[:, 1:2]
    xr = MS / jnp.maximum(x[:, 1], x[:, 3])
    yr = MS / jnp.maximum(y[:, 1], y[:, 3])
    xp = jnp.clip(jnp.round(x * xr[:, None]), 0, MS)
    yp = jnp.clip(jnp.round(y * yr[:, None]), 0, MS)
    coords = jnp.concatenate([xp, yp], axis=1).astype(f32)

    row = lambda v: v.reshape(1, -1)
    w1a = We1[:roi]
    w1b = We1[roi:roi + objc]
    w1c = We1[roi + objc:]
    wc1a = Wc1[:c]
    wc1b = Wc1[c:2 * c]
    wc1c = Wc1[2 * c:]

    obj_feats, g_obj = pl.pallas_call(
        _obj_embed_kernel,
        out_shape=[jax.ShapeDtypeStruct((n, c), f32),
                   jax.ShapeDtypeStruct((n, c), f32)],
    )(roi_features, obj_logits, bboxes,
      w1a, w1b, w1c, row(be1), We2, row(be2), Wg)

    nch = c // SC_ROW
    g_rows = g_obj.reshape(n * nch, SC_ROW)
    me = m * nch
    mep = ((me + SC_W - 1) // SC_W) * SC_W

    def _expand(col):
        e = (rel_pair_idxs[:, col:col + 1] * nch + jnp.arange(nch)).reshape(-1)
        return jnp.zeros((mep,), jnp.int32).at[:me].set(e)

    idx = jnp.concatenate([_expand(0), _expand(1)]).reshape(1, 2 * mep)
    gath = _pair_gather_sc(g_rows, idx)
    msga = gath[:me].reshape(m, c)
    msgb = gath[mep:mep + me].reshape(m, c)

    ut = jnp.transpose(union_features, (2, 3, 0, 1))
    ss, so, sbg = pl.pallas_call(
        _masked_mean_kernel,
        grid=(NSTEP,),
        in_specs=[
            pl.BlockSpec((m, 8), lambda i: (0, 0)),
            pl.BlockSpec((1, KSLAB, m, c),
                         lambda i: (i // (MS // KSLAB), i % (MS // KSLAB), 0, 0)),
        ],
        out_specs=[
            pl.BlockSpec((m, c), lambda i: (0, 0)),
            pl.BlockSpec((m, c), lambda i: (0, 0)),
            pl.BlockSpec((m, c), lambda i: (0, 0)),
        ],
        out_shape=[jax.ShapeDtypeStruct((m, c), f32)] * 3,
        scratch_shapes=[pltpu.VMEM((m, c), f32)] * 3,
    )(coords, ut)

    obj_dists, rel_dists = pl.pallas_call(
        _fuse_kernel,
        out_shape=[jax.ShapeDtypeStruct((n, objc), f32),
                   jax.ShapeDtypeStruct((m, relc), f32)],
    )(rel_pair_idxs,
      ss, so, sbg,
      obj_feats, g_obj, msga, msgb,
      Wsr, row(bsr), Wsu, row(bsu),
      Wor, row(bor), Wou, row(bou),
      Wbr, row(bbr), Wbu, row(bbu),
      wc1a, wc1b, wc1c, row(bc1), Wc2, row(bc2),
      Wg, row(bgc), Wobj, row(bobj), Wrel, row(brel))
    return (obj_dists, rel_dists)


# speedup vs baseline: 1.2416x; 1.2237x over previous
import jax
import jax.numpy as jnp
from jax import lax
from jax.experimental import pallas as pl
from jax.experimental.pallas import tpu as pltpu
from jax.experimental.pallas import tpu_sc as plsc

MS = 14
SP = MS * MS


KSLAB = 14
NSTEP = SP // KSLAB


def _tree_sum(terms):
    while len(terms) > 1:
        nxt = [a + b for a, b in zip(terms[::2], terms[1::2])]
        if len(terms) % 2:
            nxt.append(terms[-1])
        terms = nxt
    return terms[0]


def _masked_mean_kernel(coords_ref, u_ref, s_ref, o_ref, b_ref,
                        acc_s, acc_o, acc_b):
    i = pl.program_id(0)
    c = coords_ref[...]
    a0, a1 = c[:, 0:4], c[:, 4:8]
    c0, c1 = c[:, 8:12], c[:, 12:16]

    ts = acc_s[...]
    to = acc_o[...]
    tb = acc_b[...]
    zero = jnp.zeros_like(ts)
    if_first = i == 0
    ts = jnp.where(if_first, 0.0, ts)
    to = jnp.where(if_first, 0.0, to)
    tb = jnp.where(if_first, 0.0, tb)

    rf = (i // (MS // KSLAB)).astype(jnp.float32)
    cbase = (i % (MS // KSLAB)) * KSLAB
    rowin = (rf >= a0) & (rf < a1)
    cdim = ts.shape[1]
    r_i = lax.broadcasted_iota(jnp.int32, (4, 3 * cdim), 0)
    b_i = lax.broadcasted_iota(jnp.int32, (4, 3 * cdim), 1) // cdim
    sel = jnp.where(b_i <= 1, (r_i == b_i).astype(jnp.float32),
                    jnp.where(r_i <= 1, -1.0, 1.0))
    terms_s, terms_o, terms_b = [], [], []
    for k in range(KSLAB):
        cf = (cbase + k).astype(jnp.float32)
        mvec = (rowin & (cf >= c0) & (cf < c1)).astype(jnp.float32)
        bc = lax.dot_general(mvec, sel, (((1,), (0,)), ((), ())),
                             preferred_element_type=jnp.float32)
        u = u_ref[0, k]
        terms_s.append(u * bc[:, 0:cdim])
        terms_o.append(u * bc[:, cdim:2 * cdim])
        terms_b.append(u * bc[:, 2 * cdim:3 * cdim])
    ts = ts + _tree_sum(terms_s)
    to = to + _tree_sum(terms_o)
    tb = tb + _tree_sum(terms_b)

    acc_s[...] = ts
    acc_o[...] = to
    acc_b[...] = tb

    @pl.when(i == NSTEP - 1)
    def _():
        inv = 1.0 / SP
        s_ref[...] = ts * inv
        o_ref[...] = to * inv
        b_ref[...] = tb * inv


def _mm(a, b):
    return lax.dot_general(a, b, (((1,), (0,)), ((), ())),
                           preferred_element_type=jnp.float32)


def _mm_t(a, b):
    return lax.dot_general(a, b, (((0,), (0,)), ((), ())),
                           preferred_element_type=jnp.float32)


def _obj_embed_kernel(roi_ref, logits_ref, bboxes_ref,
                      w1a_ref, w1b_ref, w1c_ref, be1_ref, we2_ref, be2_ref,
                      wg_ref, objf_ref, gobj_ref):
    h1 = _mm(roi_ref[...], w1a_ref[...]) + _mm(logits_ref[...], w1b_ref[...]) \
        + _mm(bboxes_ref[...], w1c_ref[...]) + be1_ref[...]
    objf = _mm(jnp.maximum(h1, 0.0), we2_ref[...]) + be2_ref[...]
    objf_ref[...] = objf
    gobj_ref[...] = _mm(objf, wg_ref[...])


SC_W = 128
SC_LANES = 16


SC_ROW = 128


def _pair_gather_sc(g_rows, idx):
    k = idx.shape[1]
    mesh = plsc.VectorSubcoreMesh(core_axis_name="c", subcore_axis_name="s",
                                  num_cores=2, num_subcores=16)

    @pl.kernel(out_type=jax.ShapeDtypeStruct((k, SC_ROW), jnp.float32),
               mesh=mesh)
    def kern(g_hbm, idx_hbm, o_hbm):
        def body(idx_vmem, o_vmem):
            pltpu.sync_copy(g_hbm.at[idx_vmem.at[0]], o_vmem)

        pltpu.emit_pipeline(
            body,
            grid=(k // SC_W,),
            in_specs=[pl.BlockSpec((1, SC_W), lambda i: (0, i))],
            out_specs=[pl.BlockSpec((SC_W, SC_ROW), lambda i: (i, 0))],
            core_axis_name=("c", "s"),
            dimension_semantics=(pltpu.PARALLEL,),
        )(idx_hbm, o_hbm)

    return kern(g_rows, idx)


def _fuse_kernel(pairs_ref,
                 ss_ref, so_ref, sb_ref,
                 objf_ref, gobj_ref, msga_ref, msgb_ref,
                 wsr_ref, bsr_ref, wsu_ref, bsu_ref,
                 wor_ref, bor_ref, wou_ref, bou_ref,
                 wbr_ref, bbr_ref, wbu_ref, bbu_ref,
                 wc1a_ref, wc1b_ref, wc1c_ref, bc1_ref, wc2_ref, bc2_ref,
                 wg_ref, bgc_ref, wobj_ref, bobj_ref, wrel_ref, brel_ref,
                 objd_ref, reld_ref):
    f32 = jnp.float32
    mm = _mm
    mm_t = _mm_t
    obj_feats = objf_ref[...]

    def gate(s, wr, br, wu, bu):
        a = jax.nn.sigmoid(mm(jnp.maximum(mm(s, wr) + br, 0.0), wu) + bu)
        return s * a

    vs = gate(ss_ref[...], wsr_ref[...], bsr_ref[...], wsu_ref[...], bsu_ref[...])
    vo = gate(so_ref[...], wor_ref[...], bor_ref[...], wou_ref[...], bou_ref[...])
    vb = gate(sb_ref[...], wbr_ref[...], bbr_ref[...], wbu_ref[...], bbu_ref[...])

    rh = jnp.maximum(mm(vs, wc1a_ref[...]) + mm(vo, wc1b_ref[...])
                     + mm(vb, wc1c_ref[...]) + bc1_ref[...], 0.0)
    rel_feats = mm(rh, wc2_ref[...]) + bc2_ref[...]

    n = objf_ref.shape[0]
    m = rel_feats.shape[0]
    pairs = pairs_ref[...]
    obj_ids = lax.broadcasted_iota(jnp.int32, (m, n), 1)
    s_hot = (pairs[:, 0:1] == obj_ids).astype(f32)
    o_hot = (pairs[:, 1:2] == obj_ids).astype(f32)
    so = s_hot + o_hot

    g_obj = gobj_ref[...]
    g_rel = mm(rel_feats, wg_ref[...])

    a_oo = mm_t(s_hot, o_hot)
    agg_obj = mm(a_oo, g_obj) + mm_t(so, g_rel) + g_obj
    deg_obj = 1.0 + jnp.sum(a_oo, axis=1, keepdims=True) \
        + jnp.sum(so, axis=0)[:, None]
    h_obj = jnp.maximum(agg_obj / deg_obj + bgc_ref[...], 0.0)

    agg_rel = msga_ref[...] + msgb_ref[...] + g_rel
    h_rel = jnp.maximum(agg_rel * (1.0 / 3.0) + bgc_ref[...], 0.0)

    out_obj = h_obj + obj_feats
    out_rel = h_rel + rel_feats
    objd_ref[...] = mm(out_obj, wobj_ref[...]) + bobj_ref[...]
    reld_ref[...] = mm(out_rel, wrel_ref[...]) + brel_ref[...]


def kernel(roi_features, obj_logits, bboxes, union_features, rel_pair_idxs,
           We1, be1, We2, be2,
           Wsr, bsr, Wsu, bsu, Wor, bor, Wou, bou, Wbr, bbr, Wbu, bbu,
           Wc1, bc1, Wc2, bc2, Wg, bgc, Wobj, bobj, Wrel, brel):
    f32 = jnp.float32
    n, roi = roi_features.shape
    m, c = union_features.shape[0], union_features.shape[1]
    objc = obj_logits.shape[1]
    relc = Wrel.shape[1]

    sb = bboxes[rel_pair_idxs[:, 0]]
    ob = bboxes[rel_pair_idxs[:, 1]]
    pair_boxes = jnp.concatenate([sb, ob], axis=1)
    union_boxes = jnp.concatenate(
        [jnp.minimum(sb[:, :2], ob[:, :2]), jnp.maximum(sb[:, 2:], ob[:, 2:])], axis=1)
    x = pair_boxes[:, jnp.array([0, 2, 4, 6])] - union_boxes[:, 0:1]
    y = pair_boxes[:, jnp.array([1, 3, 5, 7])] - union_boxes[:, 1:2]
    xr = MS / jnp.maximum(x[:, 1], x[:, 3])
    yr = MS / jnp.maximum(y[:, 1], y[:, 3])
    xp = jnp.clip(jnp.round(x * xr[:, None]), 0, MS)
    yp = jnp.clip(jnp.round(y * yr[:, None]), 0, MS)
    neg = jnp.full((m,), -1.0)
    big = jnp.full((m,), MS + 1.0)
    a0 = jnp.stack([xp[:, 0], xp[:, 2], jnp.maximum(xp[:, 0], xp[:, 2]), neg], 1)
    a1 = jnp.stack([xp[:, 1], xp[:, 3], jnp.minimum(xp[:, 1], xp[:, 3]), big], 1)
    c0 = jnp.stack([yp[:, 0], yp[:, 2], jnp.maximum(yp[:, 0], yp[:, 2]), neg], 1)
    c1 = jnp.stack([yp[:, 1], yp[:, 3], jnp.minimum(yp[:, 1], yp[:, 3]), big], 1)
    coords = jnp.concatenate([a0, a1, c0, c1], axis=1).astype(f32)

    row = lambda v: v.reshape(1, -1)
    w1a = We1[:roi]
    w1b = We1[roi:roi + objc]
    w1c = We1[roi + objc:]
    wc1a = Wc1[:c]
    wc1b = Wc1[c:2 * c]
    wc1c = Wc1[2 * c:]

    obj_feats, g_obj = pl.pallas_call(
        _obj_embed_kernel,
        out_shape=[jax.ShapeDtypeStruct((n, c), f32),
                   jax.ShapeDtypeStruct((n, c), f32)],
    )(roi_features, obj_logits, bboxes,
      w1a, w1b, w1c, row(be1), We2, row(be2), Wg)

    nch = c // SC_ROW
    g_rows = g_obj.reshape(n * nch, SC_ROW)
    me = m * nch
    mep = ((me + SC_W - 1) // SC_W) * SC_W

    def _expand(col):
        e = (rel_pair_idxs[:, col:col + 1] * nch + jnp.arange(nch)).reshape(-1)
        return jnp.zeros((mep,), jnp.int32).at[:me].set(e)

    idx = jnp.concatenate([_expand(0), _expand(1)]).reshape(1, 2 * mep)
    gath = _pair_gather_sc(g_rows, idx)
    msga = gath[:me].reshape(m, c)
    msgb = gath[mep:mep + me].reshape(m, c)

    ut = jnp.transpose(union_features, (2, 3, 0, 1))
    ss, so, sbg = pl.pallas_call(
        _masked_mean_kernel,
        grid=(NSTEP,),
        in_specs=[
            pl.BlockSpec((m, 16), lambda i: (0, 0)),
            pl.BlockSpec((1, KSLAB, m, c),
                         lambda i: (i // (MS // KSLAB), i % (MS // KSLAB), 0, 0)),
        ],
        out_specs=[
            pl.BlockSpec((m, c), lambda i: (0, 0)),
            pl.BlockSpec((m, c), lambda i: (0, 0)),
            pl.BlockSpec((m, c), lambda i: (0, 0)),
        ],
        out_shape=[jax.ShapeDtypeStruct((m, c), f32)] * 3,
        scratch_shapes=[pltpu.VMEM((m, c), f32)] * 3,
    )(coords, ut)

    obj_dists, rel_dists = pl.pallas_call(
        _fuse_kernel,
        out_shape=[jax.ShapeDtypeStruct((n, objc), f32),
                   jax.ShapeDtypeStruct((m, relc), f32)],
    )(rel_pair_idxs,
      ss, so, sbg,
      obj_feats, g_obj, msga, msgb,
      Wsr, row(bsr), Wsu, row(bsu),
      Wor, row(bor), Wou, row(bou),
      Wbr, row(bbr), Wbu, row(bbu),
      wc1a, wc1b, wc1c, row(bc1), Wc2, row(bc2),
      Wg, row(bgc), Wobj, row(bobj), Wrel, row(brel))
    return (obj_dists, rel_dists)


# speedup vs baseline: 1.2418x; 1.0001x over previous
import jax
import jax.numpy as jnp
from jax import lax
from jax.experimental import pallas as pl
from jax.experimental.pallas import tpu as pltpu
from jax.experimental.pallas import tpu_sc as plsc

MS = 14
SP = MS * MS


KSLAB = 14
NSTEP = SP // KSLAB


def _tree_sum(terms):
    while len(terms) > 1:
        nxt = [a + b for a, b in zip(terms[::2], terms[1::2])]
        if len(terms) % 2:
            nxt.append(terms[-1])
        terms = nxt
    return terms[0]


def _masked_mean_kernel(coords_ref, u_ref, s_ref, o_ref, b_ref,
                        acc_s, acc_o, acc_b):
    i = pl.program_id(0)
    c = coords_ref[...]
    a0, a1 = c[:, 0:4], c[:, 4:8]
    c0, c1 = c[:, 8:12], c[:, 12:16]

    ts = acc_s[...]
    to = acc_o[...]
    tb = acc_b[...]
    if_first = i == 0
    ts = jnp.where(if_first, 0.0, ts)
    to = jnp.where(if_first, 0.0, to)
    tb = jnp.where(if_first, 0.0, tb)

    rf = (i // (MS // KSLAB)).astype(jnp.float32)
    cbase = (i % (MS // KSLAB)) * KSLAB
    rowin = (rf >= a0) & (rf < a1)
    cdim = ts.shape[1]
    r_i = lax.broadcasted_iota(jnp.int32, (4, 3 * cdim), 0)
    b_i = lax.broadcasted_iota(jnp.int32, (4, 3 * cdim), 1) // cdim
    sel = jnp.where(b_i <= 1, (r_i == b_i).astype(jnp.float32),
                    jnp.where(r_i <= 1, -1.0, 1.0))
    terms_s, terms_o, terms_b = [], [], []
    for k in range(KSLAB):
        cf = (cbase + k).astype(jnp.float32)
        mvec = (rowin & (cf >= c0) & (cf < c1)).astype(jnp.float32)
        bc = lax.dot_general(mvec, sel, (((1,), (0,)), ((), ())),
                             preferred_element_type=jnp.float32)
        u = u_ref[0, k]
        terms_s.append(u * bc[:, 0:cdim])
        terms_o.append(u * bc[:, cdim:2 * cdim])
        terms_b.append(u * bc[:, 2 * cdim:3 * cdim])
    ts = ts + _tree_sum(terms_s)
    to = to + _tree_sum(terms_o)
    tb = tb + _tree_sum(terms_b)

    acc_s[...] = ts
    acc_o[...] = to
    acc_b[...] = tb

    @pl.when(i == NSTEP - 1)
    def _():
        inv = 1.0 / SP
        s_ref[...] = ts * inv
        o_ref[...] = to * inv
        b_ref[...] = tb * inv


def _mm(a, b):
    return lax.dot_general(a, b, (((1,), (0,)), ((), ())),
                           preferred_element_type=jnp.float32)


def _mm_t(a, b):
    return lax.dot_general(a, b, (((0,), (0,)), ((), ())),
                           preferred_element_type=jnp.float32)


def _obj_embed_kernel(roi_ref, logits_ref, bboxes_ref,
                      w1a_ref, w1b_ref, w1c_ref, be1_ref, we2_ref, be2_ref,
                      wg_ref, objf_ref, gobj_ref):
    h1 = _mm(roi_ref[...], w1a_ref[...]) + _mm(logits_ref[...], w1b_ref[...]) \
        + _mm(bboxes_ref[...], w1c_ref[...]) + be1_ref[...]
    objf = _mm(jnp.maximum(h1, 0.0), we2_ref[...]) + be2_ref[...]
    objf_ref[...] = objf
    gobj_ref[...] = _mm(objf, wg_ref[...])


SC_W = 128
SC_ROW = 128


def _pair_gather_sc(g_rows, idx):
    k = idx.shape[1]
    mesh = plsc.VectorSubcoreMesh(core_axis_name="c", subcore_axis_name="s",
                                  num_cores=2, num_subcores=16)

    @pl.kernel(out_type=jax.ShapeDtypeStruct((k, SC_ROW), jnp.float32),
               mesh=mesh)
    def kern(g_hbm, idx_hbm, o_hbm):
        def body(idx_vmem, o_vmem):
            pltpu.sync_copy(g_hbm.at[idx_vmem.at[0]], o_vmem)

        pltpu.emit_pipeline(
            body,
            grid=(k // SC_W,),
            in_specs=[pl.BlockSpec((1, SC_W), lambda i: (0, i))],
            out_specs=[pl.BlockSpec((SC_W, SC_ROW), lambda i: (i, 0))],
            core_axis_name=("c", "s"),
            dimension_semantics=(pltpu.PARALLEL,),
        )(idx_hbm, o_hbm)

    return kern(g_rows, idx)


def _fuse_kernel(pairs_ref,
                 ss_ref, so_ref, sb_ref,
                 objf_ref, gobj_ref, msga_ref, msgb_ref,
                 wsr_ref, bsr_ref, wsu_ref, bsu_ref,
                 wor_ref, bor_ref, wou_ref, bou_ref,
                 wbr_ref, bbr_ref, wbu_ref, bbu_ref,
                 wc1a_ref, wc1b_ref, wc1c_ref, bc1_ref, wc2_ref, bc2_ref,
                 wg_ref, bgc_ref, wobj_ref, bobj_ref, wrel_ref, brel_ref,
                 objd_ref, reld_ref):
    f32 = jnp.float32
    mm = _mm
    mm_t = _mm_t
    obj_feats = objf_ref[...]

    def gate(s, wr, br, wu, bu):
        a = jax.nn.sigmoid(mm(jnp.maximum(mm(s, wr) + br, 0.0), wu) + bu)
        return s * a

    vs = gate(ss_ref[...], wsr_ref[...], bsr_ref[...], wsu_ref[...], bsu_ref[...])
    vo = gate(so_ref[...], wor_ref[...], bor_ref[...], wou_ref[...], bou_ref[...])
    vb = gate(sb_ref[...], wbr_ref[...], bbr_ref[...], wbu_ref[...], bbu_ref[...])

    rh = jnp.maximum(mm(vs, wc1a_ref[...]) + mm(vo, wc1b_ref[...])
                     + mm(vb, wc1c_ref[...]) + bc1_ref[...], 0.0)
    rel_feats = mm(rh, wc2_ref[...]) + bc2_ref[...]

    n = objf_ref.shape[0]
    m = rel_feats.shape[0]
    pairs = pairs_ref[...]
    obj_ids = lax.broadcasted_iota(jnp.int32, (m, n), 1)
    s_hot = (pairs[:, 0:1] == obj_ids).astype(f32)
    o_hot = (pairs[:, 1:2] == obj_ids).astype(f32)
    so = s_hot + o_hot

    g_obj = gobj_ref[...]
    g_rel = mm(rel_feats, wg_ref[...])

    a_oo = mm_t(s_hot, o_hot)
    agg_obj = mm(a_oo, g_obj) + mm_t(so, g_rel) + g_obj
    deg_obj = 1.0 + jnp.sum(a_oo, axis=1, keepdims=True) \
        + jnp.sum(so, axis=0)[:, None]
    h_obj = jnp.maximum(agg_obj / deg_obj + bgc_ref[...], 0.0)

    agg_rel = msga_ref[...] + msgb_ref[...] + g_rel
    h_rel = jnp.maximum(agg_rel * (1.0 / 3.0) + bgc_ref[...], 0.0)

    out_obj = h_obj + obj_feats
    out_rel = h_rel + rel_feats
    objd_ref[...] = mm(out_obj, wobj_ref[...]) + bobj_ref[...]
    reld_ref[...] = mm(out_rel, wrel_ref[...]) + brel_ref[...]


def kernel(roi_features, obj_logits, bboxes, union_features, rel_pair_idxs,
           We1, be1, We2, be2,
           Wsr, bsr, Wsu, bsu, Wor, bor, Wou, bou, Wbr, bbr, Wbu, bbu,
           Wc1, bc1, Wc2, bc2, Wg, bgc, Wobj, bobj, Wrel, brel):
    f32 = jnp.float32
    n, roi = roi_features.shape
    m, c = union_features.shape[0], union_features.shape[1]
    objc = obj_logits.shape[1]
    relc = Wrel.shape[1]

    sb = bboxes[rel_pair_idxs[:, 0]]
    ob = bboxes[rel_pair_idxs[:, 1]]
    pair_boxes = jnp.concatenate([sb, ob], axis=1)
    union_boxes = jnp.concatenate(
        [jnp.minimum(sb[:, :2], ob[:, :2]), jnp.maximum(sb[:, 2:], ob[:, 2:])], axis=1)
    x = pair_boxes[:, jnp.array([0, 2, 4, 6])] - union_boxes[:, 0:1]
    y = pair_boxes[:, jnp.array([1, 3, 5, 7])] - union_boxes[:, 1:2]
    xr = MS / jnp.maximum(x[:, 1], x[:, 3])
    yr = MS / jnp.maximum(y[:, 1], y[:, 3])
    xp = jnp.clip(jnp.round(x * xr[:, None]), 0, MS)
    yp = jnp.clip(jnp.round(y * yr[:, None]), 0, MS)
    neg = jnp.full((m,), -1.0)
    big = jnp.full((m,), MS + 1.0)
    a0 = jnp.stack([xp[:, 0], xp[:, 2], jnp.maximum(xp[:, 0], xp[:, 2]), neg], 1)
    a1 = jnp.stack([xp[:, 1], xp[:, 3], jnp.minimum(xp[:, 1], xp[:, 3]), big], 1)
    c0 = jnp.stack([yp[:, 0], yp[:, 2], jnp.maximum(yp[:, 0], yp[:, 2]), neg], 1)
    c1 = jnp.stack([yp[:, 1], yp[:, 3], jnp.minimum(yp[:, 1], yp[:, 3]), big], 1)
    coords = jnp.concatenate([a0, a1, c0, c1], axis=1).astype(f32)

    row = lambda v: v.reshape(1, -1)
    w1a = We1[:roi]
    w1b = We1[roi:roi + objc]
    w1c = We1[roi + objc:]
    wc1a = Wc1[:c]
    wc1b = Wc1[c:2 * c]
    wc1c = Wc1[2 * c:]

    obj_feats, g_obj = pl.pallas_call(
        _obj_embed_kernel,
        out_shape=[jax.ShapeDtypeStruct((n, c), f32),
                   jax.ShapeDtypeStruct((n, c), f32)],
    )(roi_features, obj_logits, bboxes,
      w1a, w1b, w1c, row(be1), We2, row(be2), Wg)

    nch = c // SC_ROW
    g_rows = g_obj.reshape(n * nch, SC_ROW)
    me = m * nch
    mep = ((me + SC_W - 1) // SC_W) * SC_W

    def _expand(col):
        e = (rel_pair_idxs[:, col:col + 1] * nch + jnp.arange(nch)).reshape(-1)
        return jnp.zeros((mep,), jnp.int32).at[:me].set(e)

    idx = jnp.concatenate([_expand(0), _expand(1)]).reshape(1, 2 * mep)
    gath = _pair_gather_sc(g_rows, idx)
    msga = gath[:me].reshape(m, c)
    msgb = gath[mep:mep + me].reshape(m, c)

    ut = jnp.transpose(union_features, (2, 3, 0, 1))
    ss, so, sbg = pl.pallas_call(
        _masked_mean_kernel,
        grid=(NSTEP,),
        in_specs=[
            pl.BlockSpec((m, 16), lambda i: (0, 0)),
            pl.BlockSpec((1, KSLAB, m, c),
                         lambda i: (i // (MS // KSLAB), i % (MS // KSLAB), 0, 0)),
        ],
        out_specs=[
            pl.BlockSpec((m, c), lambda i: (0, 0)),
            pl.BlockSpec((m, c), lambda i: (0, 0)),
            pl.BlockSpec((m, c), lambda i: (0, 0)),
        ],
        out_shape=[jax.ShapeDtypeStruct((m, c), f32)] * 3,
        scratch_shapes=[pltpu.VMEM((m, c), f32)] * 3,
    )(coords, ut)

    obj_dists, rel_dists = pl.pallas_call(
        _fuse_kernel,
        out_shape=[jax.ShapeDtypeStruct((n, objc), f32),
                   jax.ShapeDtypeStruct((m, relc), f32)],
    )(rel_pair_idxs,
      ss, so, sbg,
      obj_feats, g_obj, msga, msgb,
      Wsr, row(bsr), Wsu, row(bsu),
      Wor, row(bor), Wou, row(bou),
      Wbr, row(bbr), Wbu, row(bbu),
      wc1a, wc1b, wc1c, row(bc1), Wc2, row(bc2),
      Wg, row(bgc), Wobj, row(bobj), Wrel, row(brel))
    return (obj_dists, rel_dists)
